# baseline (speedup 1.0000x reference)
import functools

import jax
import jax.numpy as jnp
from jax.experimental import pallas as pl
from jax.experimental.pallas import tpu as pltpu

_D = 128
_BLK = 1000


def _update_body(glob_ref, wcat_ref, w3_ref, ball_ref, xf_ref, y_ref, z1_ref,
                 deg_ref, out_ref):
    xf = xf_ref[...]
    cat = jnp.concatenate(
        [xf, y_ref[...], xf * deg_ref[...], z1_ref[...]], axis=1)
    acc = jax.lax.dot_general(cat, wcat_ref[...], (((1,), (0,)), ((), ())),
                              preferred_element_type=jnp.float32)
    cvec = jax.lax.dot_general(glob_ref[...], w3_ref[...],
                               (((1,), (0,)), ((), ())),
                               preferred_element_type=jnp.float32)
    out_ref[...] = acc + cvec + ball_ref[...]


def _update(xf, y, z1, deg, glob, wcat, w3, ball):
    rows = xf.shape[0]
    assert rows % _BLK == 0
    return pl.pallas_call(
        _update_body,
        grid=(rows // _BLK,),
        in_specs=[
            pl.BlockSpec((1, _D), lambda i: (0, 0)),
            pl.BlockSpec((4 * _D, _D), lambda i: (0, 0)),
            pl.BlockSpec((_D, _D), lambda i: (0, 0)),
            pl.BlockSpec((1, _D), lambda i: (0, 0)),
            pl.BlockSpec((_BLK, _D), lambda i: (i, 0)),
            pl.BlockSpec((_BLK, _D), lambda i: (i, 0)),
            pl.BlockSpec((_BLK, _D), lambda i: (i, 0)),
            pl.BlockSpec((_BLK, 1), lambda i: (i, 0)),
        ],
        out_specs=pl.BlockSpec((_BLK, _D), lambda i: (i, 0)),
        out_shape=jax.ShapeDtypeStruct((rows, _D), jnp.float32),
    )(glob, wcat, w3, ball, xf, y, z1, deg)


def kernel(x_g, x_lg, edge_index_g, edge_index_lg, edge_index_glg,
           Wt_main, bt_main, Wt_list, bt_list,
           Wg_main, bg_main, Wg_list, bg_list):
    n = x_g.shape[0]
    m = x_lg.shape[0]

    def agg2(x, ei):
        src, dst = ei[0], ei[1]
        z1 = jnp.zeros_like(x).at[dst].add(x[src])
        xf = jnp.zeros_like(x).at[dst].add(z1[src])
        return z1, xf

    z1_g, xf_g = agg2(x_g, edge_index_g)
    z1_lg, xf_lg = agg2(x_lg, edge_index_lg)

    deg_g = jnp.zeros((n,), jnp.float32).at[edge_index_g[1]].add(1.0)
    deg_lg = jnp.zeros((m,), jnp.float32).at[edge_index_lg[1]].add(1.0)

    x_comb = jnp.concatenate([xf_g, xf_lg], axis=0)
    y = jnp.zeros_like(x_comb).at[edge_index_glg[1]].add(
        x_comb[edge_index_glg[0]])

    glob_g = jnp.mean(xf_g, axis=0, keepdims=True)
    glob_lg = jnp.mean(xf_lg, axis=0, keepdims=True)

    wcat_t = jnp.concatenate(
        [Wt_main[0] + Wt_list[1], Wt_main[1], Wt_main[2], Wt_list[0]], axis=0)
    ball_t = (bt_main.sum(0) + bt_list.sum(0))[None, :]
    wcat_g = jnp.concatenate(
        [Wg_main[0] + Wg_list[1], Wg_main[1], Wg_main[2], Wg_list[0]], axis=0)
    ball_g = (bg_main.sum(0) + bg_list.sum(0))[None, :]

    out_g = _update(xf_g, y[:n], z1_g, deg_g[:, None], glob_g, wcat_t,
                    Wt_main[3], ball_t)
    out_lg = _update(xf_lg, y[n:], z1_lg, deg_lg[:, None], glob_lg, wcat_g,
                     Wg_main[3], ball_g)
    return (out_g, out_lg)
